# baseline (speedup 1.0000x reference)
import functools

import jax
import jax.numpy as jnp
from jax import lax
from jax.experimental import pallas as pl
from jax.experimental.pallas import tpu as pltpu
from jax.experimental.pallas import tpu_sc as plsc

VOCAB = 100000
HIDDEN = 768
MAX_POS = 512
BATCH = 1024
SEQ = 512

B = BATCH * SEQ
NC, NS = 2, 16
NW = NC * NS
BPW = B // NW
SPW = BPW // SEQ
S = 8
C = 4
RPC = S * C
NPH = SEQ // C
SGP = SPW // S
NCH = NPH * SGP
NVEC = HIDDEN // 16
NB = 4
ITERS = (NCH + NB - 1) // NB
IQ = NCH // 4


def _body(ids_hbm, word_hbm, pos_hbm, out_hbm,
          idx_v, pos_v, b0, b1, b2, b3, gsem, osem):
    bufs = (b0, b1, b2, b3)
    wid = lax.axis_index("s") * NC + lax.axis_index("c")
    pltpu.sync_copy(ids_hbm.at[wid, pl.ds(0, IQ)], idx_v)

    def gather_start(chunk, b):
        pltpu.async_copy(word_hbm.at[idx_v.at[lax.rem(chunk, IQ)]], bufs[b],
                         gsem.at[b])

    def gather_wait(b):
        pltpu.make_async_copy(word_hbm.at[idx_v.at[0]], bufs[b],
                              gsem.at[b]).wait()

    def out_start(chunk, b):
        p = chunk // SGP
        sg = lax.rem(chunk, SGP)
        q0 = p * C
        for s in range(S):
            pltpu.async_copy(bufs[b].at[pl.ds(s * C, C)],
                             out_hbm.at[wid * SPW + sg * S + s, pl.ds(q0, C)],
                             osem.at[b])

    def out_wait(b):
        for _ in range(S):
            pltpu.make_async_copy(bufs[b].at[pl.ds(0, C)],
                                  out_hbm.at[0, pl.ds(0, C)], osem.at[b]).wait()

    def compute(b, chunk):
        sg = lax.rem(chunk, SGP)

        @pl.when(sg == 0)
        def _():
            p = chunk // SGP
            pltpu.sync_copy(pos_hbm.at[pl.ds(p * C, C)], pos_v)

        word = bufs[b]

        @plsc.parallel_loop(0, NVEC, unroll=2)
        def jloop(j):
            off = pl.multiple_of(j * 16, 16)
            sl = pl.ds(off, 16)
            pv = [pos_v[r, sl] for r in range(C)]
            for s in range(S):
                for r in range(C):
                    row = s * C + r
                    word[row, sl] = word[row, sl] + pv[r]

    def step(kk, carry):
        for b in range(NB):
            k = kk * NB + b
            kmod = lax.rem(k, NCH)
            nb = (b + 1) % NB
            gather_wait(b)

            @pl.when(k >= NB - 1)
            def _():
                out_wait(nb)

            @pl.when(lax.rem(k + 1, IQ) == 0)
            def _():
                q = lax.rem((k + 1) // IQ, NCH // IQ)
                pltpu.sync_copy(ids_hbm.at[wid, pl.ds(q * IQ, IQ)], idx_v)

            gather_start(lax.rem(k + 1, NCH), nb)
            compute(b, kmod)
            out_start(kmod, b)
        return carry

    gather_start(0, 0)
    lax.fori_loop(0, ITERS, step, 0)
    gather_wait((ITERS * NB) % NB)
    for b in range(1, NB):
        out_wait(b)


@jax.jit
def _run(ids_perm, word_embeddings, position_embeddings):
    mesh = plsc.VectorSubcoreMesh(core_axis_name="c", subcore_axis_name="s")
    f = functools.partial(
        pl.kernel,
        mesh=mesh,
        out_type=jax.ShapeDtypeStruct((BATCH, SEQ, HIDDEN), jnp.float32),
        scratch_types=[
            pltpu.VMEM((IQ, RPC), jnp.int32),
            pltpu.VMEM((C, HIDDEN), jnp.float32),
            pltpu.VMEM((RPC, HIDDEN), jnp.float32),
            pltpu.VMEM((RPC, HIDDEN), jnp.float32),
            pltpu.VMEM((RPC, HIDDEN), jnp.float32),
            pltpu.VMEM((RPC, HIDDEN), jnp.float32),
            pltpu.SemaphoreType.DMA((NB,)),
            pltpu.SemaphoreType.DMA((NB,)),
        ],
    )(_body)
    return f(ids_perm, word_embeddings, position_embeddings)


def kernel(input_ids, word_embeddings, position_embeddings):
    arr = input_ids.astype(jnp.int32).reshape(NW, SGP, S, NPH, C)
    ids_perm = arr.transpose(0, 3, 1, 2, 4).reshape(NW, NCH, RPC)
    return _run(ids_perm, word_embeddings, position_embeddings)


# speedup vs baseline: 1.3161x; 1.3161x over previous
import functools

import jax
import jax.numpy as jnp
from jax import lax
from jax.experimental import pallas as pl
from jax.experimental.pallas import tpu as pltpu
from jax.experimental.pallas import tpu_sc as plsc

VOCAB = 100000
HIDDEN = 768
MAX_POS = 512
BATCH = 1024
SEQ = 512

B = BATCH * SEQ
NC, NS = 2, 16
NW = NC * NS
BPW = B // NW
SPW = BPW // SEQ
S = 4
C = 8
RPC = S * C
NPH = SEQ // C
SGP = SPW // S
NCH = NPH * SGP
NVEC = HIDDEN // 16
NB = 3
ITERS = (NCH + NB - 1) // NB
IQ = NCH // 4


def _body(ids_hbm, word_hbm, pos_hbm, out_hbm,
          idx_v, pos_v, b0, b1, b2, gsem, osem):
    bufs = (b0, b1, b2)
    wid = lax.axis_index("s") * NC + lax.axis_index("c")
    pltpu.sync_copy(ids_hbm.at[wid, pl.ds(0, IQ)], idx_v)

    def gather_start(chunk, b):
        pltpu.async_copy(word_hbm.at[idx_v.at[lax.rem(chunk, IQ)]], bufs[b],
                         gsem.at[b])

    def gather_wait(b):
        pltpu.make_async_copy(word_hbm.at[idx_v.at[0]], bufs[b],
                              gsem.at[b]).wait()

    def out_start(chunk, b):
        p = chunk // SGP
        sg = lax.rem(chunk, SGP)
        q0 = p * C
        for s in range(S):
            pltpu.async_copy(bufs[b].at[pl.ds(s * C, C)],
                             out_hbm.at[wid * SPW + sg * S + s, pl.ds(q0, C)],
                             osem.at[b])

    def out_wait(b):
        for _ in range(S):
            pltpu.make_async_copy(bufs[b].at[pl.ds(0, C)],
                                  out_hbm.at[0, pl.ds(0, C)], osem.at[b]).wait()

    def compute(b, chunk):
        sg = lax.rem(chunk, SGP)

        @pl.when(sg == 0)
        def _():
            p = chunk // SGP
            pltpu.sync_copy(pos_hbm.at[pl.ds(p * C, C)], pos_v)

        word = bufs[b]

        @plsc.parallel_loop(0, NVEC)
        def jloop(j):
            off = pl.multiple_of(j * 16, 16)
            sl = pl.ds(off, 16)
            pv = [pos_v[r, sl] for r in range(C)]
            for s in range(S):
                for r in range(C):
                    row = s * C + r
                    word[row, sl] = word[row, sl] + pv[r]

    def step(kk, carry):
        for b in range(NB):
            k = kk * NB + b
            kmod = lax.rem(k, NCH)
            nb = (b + 1) % NB
            gather_wait(b)

            @pl.when(k >= NB - 1)
            def _():
                out_wait(nb)

            @pl.when(lax.rem(k + 1, IQ) == 0)
            def _():
                q = lax.rem((k + 1) // IQ, NCH // IQ)
                pltpu.sync_copy(ids_hbm.at[wid, pl.ds(q * IQ, IQ)], idx_v)

            gather_start(lax.rem(k + 1, NCH), nb)
            compute(b, kmod)
            out_start(kmod, b)
        return carry

    gather_start(0, 0)
    lax.fori_loop(0, ITERS, step, 0)
    gather_wait((ITERS * NB) % NB)
    for b in range(1, NB):
        out_wait(b)


@jax.jit
def _run(ids_perm, word_embeddings, position_embeddings):
    mesh = plsc.VectorSubcoreMesh(core_axis_name="c", subcore_axis_name="s")
    f = functools.partial(
        pl.kernel,
        mesh=mesh,
        out_type=jax.ShapeDtypeStruct((BATCH, SEQ, HIDDEN), jnp.float32),
        scratch_types=[
            pltpu.VMEM((IQ, RPC), jnp.int32),
            pltpu.VMEM((C, HIDDEN), jnp.float32),
            pltpu.VMEM((RPC, HIDDEN), jnp.float32),
            pltpu.VMEM((RPC, HIDDEN), jnp.float32),
            pltpu.VMEM((RPC, HIDDEN), jnp.float32),
            pltpu.SemaphoreType.DMA((NB,)),
            pltpu.SemaphoreType.DMA((NB,)),
        ],
    )(_body)
    return f(ids_perm, word_embeddings, position_embeddings)


def kernel(input_ids, word_embeddings, position_embeddings):
    arr = input_ids.astype(jnp.int32).reshape(NW, SGP, S, NPH, C)
    ids_perm = arr.transpose(0, 3, 1, 2, 4).reshape(NW, NCH, RPC)
    return _run(ids_perm, word_embeddings, position_embeddings)
